# baseline (speedup 1.0000x reference)
import jax
import jax.numpy as jnp
from jax import lax
from jax.experimental import pallas as pl
from jax.experimental.pallas import tpu as pltpu

_BLOCK_ROWS = 1024


def _binarize_pack_kernel(x_ref, m_ref, o_ref):
    m = m_ref[...]
    c = jnp.logical_and(m > 0.0, x_ref[...] >= m)
    br, f = x_ref.shape
    row_bit = lax.broadcasted_iota(jnp.int32, (br, f), 0) % 32
    w = jnp.where(c, jnp.int32(1) << row_bit, jnp.int32(0))
    o_ref[...] = jnp.sum(w.reshape(br // 32, 32, f), axis=1)


def kernel(inputs, medians):
    n, f = inputs.shape
    m2 = medians.reshape(1, f)
    grid = (n // _BLOCK_ROWS,)
    packed = pl.pallas_call(
        _binarize_pack_kernel,
        grid=grid,
        in_specs=[
            pl.BlockSpec((_BLOCK_ROWS, f), lambda i: (i, 0)),
            pl.BlockSpec((1, f), lambda i: (0, 0)),
        ],
        out_specs=pl.BlockSpec((_BLOCK_ROWS // 32, f), lambda i: (i, 0)),
        out_shape=jax.ShapeDtypeStruct((n // 32, f), jnp.int32),
        compiler_params=pltpu.CompilerParams(
            dimension_semantics=("parallel",),
        ),
    )(inputs, m2)
    bits = jnp.arange(32, dtype=jnp.uint32)
    out3 = (packed.astype(jnp.uint32)[:, None, :] >> bits[None, :, None]) & jnp.uint32(1)
    return (out3 != 0).reshape(n, f)


# speedup vs baseline: 1.0024x; 1.0024x over previous
import jax
import jax.numpy as jnp
from jax import lax
from jax.experimental import pallas as pl
from jax.experimental.pallas import tpu as pltpu

_BLOCK_ROWS = 1024


def _binarize_pack_kernel(x_ref, m_ref, o_ref):
    m = m_ref[...]
    c = jnp.logical_and(m > 0.0, x_ref[...] >= m)
    br, f = x_ref.shape
    row_bit = lax.broadcasted_iota(jnp.int32, (br, f), 0) % 32
    w = jnp.where(c, jnp.int32(1) << row_bit, jnp.int32(0))
    o_ref[...] = jnp.sum(w.reshape(br // 32, 32, f), axis=1)


def kernel(inputs, medians):
    n, f = inputs.shape
    m2 = medians.reshape(1, f)
    grid = (n // _BLOCK_ROWS,)
    packed = pl.pallas_call(
        _binarize_pack_kernel,
        grid=grid,
        in_specs=[
            pl.BlockSpec((_BLOCK_ROWS, f), lambda i: (i, 0)),
            pl.BlockSpec((1, f), lambda i: (0, 0)),
        ],
        out_specs=pl.BlockSpec((_BLOCK_ROWS // 32, f), lambda i: (i, 0)),
        out_shape=jax.ShapeDtypeStruct((n // 32, f), jnp.int32),
        compiler_params=pltpu.CompilerParams(
            dimension_semantics=("parallel",),
        ),
    )(inputs, m2)
    p2 = jnp.repeat(packed.astype(jnp.uint32), 32, axis=0)
    row_bit = (jnp.arange(n, dtype=jnp.uint32) % 32)[:, None]
    return ((p2 >> row_bit) & jnp.uint32(1)) != 0


# speedup vs baseline: 1.7171x; 1.7130x over previous
import jax
import jax.numpy as jnp
from jax.experimental import pallas as pl
from jax.experimental.pallas import tpu as pltpu

_BLOCK_ROWS = 1024


def _binarize_kernel(x_ref, m_ref, o_ref):
    m = m_ref[...]
    c = jnp.logical_and(m > 0.0, x_ref[...] >= m)
    o_ref[...] = jnp.where(c, jnp.int32(1), jnp.int32(0)).astype(jnp.int4)


def kernel(inputs, medians):
    n, f = inputs.shape
    m2 = medians.reshape(1, f)
    grid = (n // _BLOCK_ROWS,)
    out_i4 = pl.pallas_call(
        _binarize_kernel,
        grid=grid,
        in_specs=[
            pl.BlockSpec((_BLOCK_ROWS, f), lambda i: (i, 0)),
            pl.BlockSpec((1, f), lambda i: (0, 0)),
        ],
        out_specs=pl.BlockSpec((_BLOCK_ROWS, f), lambda i: (i, 0)),
        out_shape=jax.ShapeDtypeStruct((n, f), jnp.int4),
        compiler_params=pltpu.CompilerParams(
            dimension_semantics=("parallel",),
        ),
    )(inputs, m2)
    return out_i4.astype(jnp.bool_)
